# baseline (speedup 1.0000x reference)
import functools

import jax
import jax.numpy as jnp
from jax import lax
from jax.experimental import pallas as pl
from jax.experimental.pallas import tpu as pltpu
from jax.experimental.pallas import tpu_sc as plsc

N = 10000
E = 320000
D = 128
NW = 32
CH = 128
NCH = 80
EPW = NCH * CH
EP = NW * EPW
NPAD = 10240
RPT = NPAD // 16
BR = 2000

_mesh = plsc.VectorSubcoreMesh(core_axis_name="c", subcore_axis_name="s")
_sc_params = pltpu.CompilerParams(use_tc_tiling_on_sc=False)


@functools.partial(
    pl.kernel,
    mesh=_mesh,
    out_type=jax.ShapeDtypeStruct((2, NPAD), jnp.float32),
    scratch_types=[
        pltpu.VMEM((NCH, CH), jnp.int32),
        pltpu.VMEM((CH,), jnp.float32),
        pltpu.VMEM((RPT,), jnp.float32),
        pltpu.VMEM_SHARED((NPAD,), jnp.float32),
        pltpu.SemaphoreType.DMA,
    ],
    compiler_params=_sc_params,
)
def _deg_sc(col_hbm, out_hbm, colv, ones_v, bounce, dacc, sem):
    c = lax.axis_index("c")
    s = lax.axis_index("s")
    wid = c * 16 + s
    pltpu.sync_copy(col_hbm.at[wid], colv)

    def fill_ones(i, _):
        ones_v[pl.ds(i * 16, 16)] = jnp.ones((16,), jnp.float32)
        return 0
    lax.fori_loop(0, CH // 16, fill_ones, 0)

    def fill_zero(i, _):
        bounce[pl.ds(i * 16, 16)] = jnp.zeros((16,), jnp.float32)
        return 0
    lax.fori_loop(0, RPT // 16, fill_zero, 0)

    pltpu.sync_copy(bounce, dacc.at[pl.ds(s * RPT, RPT)])
    plsc.subcore_barrier()

    def body(j, _):
        pltpu.async_copy(ones_v, dacc.at[colv.at[j]], sem, add=True)
        return 0
    lax.fori_loop(0, NCH, body, 0)

    def drain(j, _):
        pltpu.make_async_copy(ones_v, dacc.at[colv.at[j]], sem).wait()
        return 0
    lax.fori_loop(0, NCH, drain, 0)

    plsc.subcore_barrier()
    pltpu.sync_copy(dacc.at[pl.ds(s * RPT, RPT)], bounce)
    pltpu.sync_copy(bounce, out_hbm.at[c, pl.ds(s * RPT, RPT)])


DH = D // 2
NCH2 = 160
EPT = NCH2 * CH


NBUF = 5
LOOK = 2


@functools.partial(
    pl.kernel,
    mesh=_mesh,
    out_type=jax.ShapeDtypeStruct((2, NPAD, DH), jnp.float32),
    scratch_types=[
        pltpu.VMEM((NCH2, CH), jnp.int32),
        pltpu.VMEM((NCH2, CH), jnp.int32),
        [pltpu.VMEM((CH, DH), jnp.float32)] * NBUF,
        [pltpu.SemaphoreType.DMA] * NBUF,
        [pltpu.SemaphoreType.DMA] * NBUF,
        pltpu.VMEM((CH,), jnp.int32),
        pltpu.VMEM_SHARED((NPAD, DH), jnp.float32),
    ],
    compiler_params=_sc_params,
)
def _edge_sc(y_hbm, row_hbm, col_hbm, out_hbm,
             rowv, colv, bufs, gsems, ssems, yidx, acc):
    c = lax.axis_index("c")
    s = lax.axis_index("s")
    pltpu.sync_copy(row_hbm.at[c, s], rowv)
    pltpu.sync_copy(col_hbm.at[s], colv)

    lanes = jnp.arange(16, dtype=jnp.int32)
    for k in range(RPT // CH):
        r0 = s * RPT + k * CH
        for q in range(CH // 16):
            base = 2 * (r0 + 16 * q) + c
            yidx[pl.ds(16 * q, 16)] = jnp.minimum(lanes * 2 + base, 2 * N - 1)
        pltpu.async_copy(y_hbm.at[yidx], bufs[0], gsems[0]).wait()
        pltpu.sync_copy(bufs[0], acc.at[pl.ds(r0, CH)])
    plsc.subcore_barrier()

    for b in range(LOOK):
        pltpu.async_copy(y_hbm.at[rowv.at[b]], bufs[b], gsems[b])

    def group(g, _):
        for b in range(NBUF):
            j = NBUF * g + b
            bb = (b + LOOK) % NBUF

            @pl.when(j + LOOK < NCH2)
            def _():
                @pl.when(j + LOOK >= NBUF)
                def _():
                    pltpu.make_async_copy(
                        bufs[bb], acc.at[colv.at[j + LOOK - NBUF]],
                        ssems[bb]).wait()
                pltpu.async_copy(y_hbm.at[rowv.at[j + LOOK]], bufs[bb], gsems[bb])

            pltpu.make_async_copy(y_hbm.at[rowv.at[j]], bufs[b], gsems[b]).wait()
            pltpu.async_copy(bufs[b], acc.at[colv.at[j]], ssems[b], add=True)
        return 0
    lax.fori_loop(0, NCH2 // NBUF, group, 0)

    for j in range(NCH2 - NBUF, NCH2):
        pltpu.make_async_copy(bufs[j % NBUF], acc.at[colv.at[j]],
                              ssems[j % NBUF]).wait()

    plsc.subcore_barrier()

    for k in range(RPT // CH):
        r0 = s * RPT + k * CH
        pltpu.sync_copy(acc.at[pl.ds(r0, CH)], bufs[0])
        pltpu.sync_copy(bufs[0], out_hbm.at[c, pl.ds(r0, CH)])


def _mm_body(emb_ref, w_ref, x_ref):
    x_ref[...] = jnp.dot(emb_ref[...], w_ref[...],
                         preferred_element_type=jnp.float32)


def _scale_body(x_ref, hist_ref, y_ref):
    deg = hist_ref[:, 0] + hist_ref[:, 1] + 1.0
    dinv = lax.rsqrt(deg)
    y_ref[...] = x_ref[...] * dinv[:, None]


def _fin_body(p_ref, hist_ref, b_ref, o_ref):
    deg = hist_ref[:, 0] + hist_ref[:, 1] + 1.0
    dinv = lax.rsqrt(deg)
    ssum = jnp.concatenate([p_ref[0], p_ref[1]], axis=1)
    o_ref[...] = jnp.maximum(ssum * dinv[:, None] + b_ref[...], 0.0)


def kernel(node_emb, edge_index, W, b):
    row = edge_index[0].astype(jnp.int32)
    col = edge_index[1].astype(jnp.int32)
    npd = EP - E
    ar = jnp.arange(npd, dtype=jnp.int32)
    row_flat = jnp.concatenate([row, (ar * 131) % N])
    col_flat = jnp.concatenate([col, N + ar % (NPAD - N)])
    col_p = col_flat.reshape(NW, NCH, CH)
    col16 = col_flat.reshape(16, NCH2, CH)
    row16 = row_flat.reshape(16, NCH2, CH)
    row4 = 2 * row16[None] + jnp.arange(2, dtype=jnp.int32)[:, None, None, None]

    hist = _deg_sc(col_p)
    hist_t = jnp.swapaxes(hist, 0, 1)

    x = pl.pallas_call(
        _mm_body,
        grid=(N // BR,),
        in_specs=[
            pl.BlockSpec((BR, D), lambda i: (i, 0)),
            pl.BlockSpec((D, D), lambda i: (0, 0)),
        ],
        out_specs=pl.BlockSpec((BR, D), lambda i: (i, 0)),
        out_shape=jax.ShapeDtypeStruct((N, D), jnp.float32),
    )(node_emb, W)

    y = pl.pallas_call(
        _scale_body,
        grid=(N // BR,),
        in_specs=[
            pl.BlockSpec((BR, D), lambda i: (i, 0)),
            pl.BlockSpec((BR, 2), lambda i: (i, 0)),
        ],
        out_specs=pl.BlockSpec((BR, D), lambda i: (i, 0)),
        out_shape=jax.ShapeDtypeStruct((N, D), jnp.float32),
    )(x, hist_t)

    y2f = y.reshape(2 * N, DH)
    p = _edge_sc(y2f, row4, col16)

    out = pl.pallas_call(
        _fin_body,
        grid=(N // BR,),
        in_specs=[
            pl.BlockSpec((2, BR, DH), lambda i: (0, i, 0)),
            pl.BlockSpec((BR, 2), lambda i: (i, 0)),
            pl.BlockSpec((1, D), lambda i: (0, 0)),
        ],
        out_specs=pl.BlockSpec((BR, D), lambda i: (i, 0)),
        out_shape=jax.ShapeDtypeStruct((N, D), jnp.float32),
    )(p, hist_t, b.reshape(1, D))
    return out


# speedup vs baseline: 1.0677x; 1.0677x over previous
import functools

import jax
import jax.numpy as jnp
from jax import lax
from jax.experimental import pallas as pl
from jax.experimental.pallas import tpu as pltpu
from jax.experimental.pallas import tpu_sc as plsc

N = 10000
E = 320000
D = 128
NW = 32
CH = 128
NCH = 80
EPW = NCH * CH
EP = NW * EPW
NPAD = 10240
RPT = NPAD // 16
BR = 2000

_mesh = plsc.VectorSubcoreMesh(core_axis_name="c", subcore_axis_name="s")
_sc_params = pltpu.CompilerParams(use_tc_tiling_on_sc=False)


@functools.partial(
    pl.kernel,
    mesh=_mesh,
    out_type=jax.ShapeDtypeStruct((2, NPAD), jnp.float32),
    scratch_types=[
        pltpu.VMEM((NCH, CH), jnp.int32),
        pltpu.VMEM((CH,), jnp.float32),
        pltpu.VMEM((RPT,), jnp.float32),
        pltpu.VMEM_SHARED((NPAD,), jnp.float32),
        pltpu.SemaphoreType.DMA,
    ],
    compiler_params=_sc_params,
)
def _deg_sc(col_hbm, out_hbm, colv, ones_v, bounce, dacc, sem):
    c = lax.axis_index("c")
    s = lax.axis_index("s")
    wid = c * 16 + s
    pltpu.sync_copy(col_hbm.at[wid], colv)

    def fill_ones(i, _):
        ones_v[pl.ds(i * 16, 16)] = jnp.ones((16,), jnp.float32)
        return 0
    lax.fori_loop(0, CH // 16, fill_ones, 0)

    def fill_zero(i, _):
        bounce[pl.ds(i * 16, 16)] = jnp.zeros((16,), jnp.float32)
        return 0
    lax.fori_loop(0, RPT // 16, fill_zero, 0)

    pltpu.sync_copy(bounce, dacc.at[pl.ds(s * RPT, RPT)])
    plsc.subcore_barrier()

    def body(j, _):
        pltpu.async_copy(ones_v, dacc.at[colv.at[j]], sem, add=True)
        return 0
    lax.fori_loop(0, NCH, body, 0)

    def drain(j, _):
        pltpu.make_async_copy(ones_v, dacc.at[colv.at[j]], sem).wait()
        return 0
    lax.fori_loop(0, NCH, drain, 0)

    plsc.subcore_barrier()
    pltpu.sync_copy(dacc.at[pl.ds(s * RPT, RPT)], bounce)
    pltpu.sync_copy(bounce, out_hbm.at[c, pl.ds(s * RPT, RPT)])


DH = D // 2
NCH2 = 160
EPT = NCH2 * CH


NBUF = 5
LOOK = 2


@functools.partial(
    pl.kernel,
    mesh=_mesh,
    out_type=jax.ShapeDtypeStruct((2, NPAD, DH), jnp.float32),
    scratch_types=[
        pltpu.VMEM((NCH2, CH), jnp.int32),
        pltpu.VMEM((NCH2, CH), jnp.int32),
        [pltpu.VMEM((CH, DH), jnp.float32)] * NBUF,
        [pltpu.SemaphoreType.DMA] * NBUF,
        [pltpu.SemaphoreType.DMA] * NBUF,
        pltpu.VMEM_SHARED((NPAD, DH), jnp.float32),
    ],
    compiler_params=_sc_params,
)
def _edge_sc(y_hbm, row_hbm, col_hbm, out_hbm,
             rowv, colv, bufs, gsems, ssems, acc):
    c = lax.axis_index("c")
    s = lax.axis_index("s")
    pltpu.sync_copy(row_hbm.at[c, s], rowv)
    pltpu.sync_copy(col_hbm.at[s], colv)

    def zrow(i, _):
        def zcol(jj, _2):
            bufs[0][i, pl.ds(jj * 16, 16)] = jnp.zeros((16,), jnp.float32)
            return 0
        lax.fori_loop(0, DH // 16, zcol, 0)
        return 0
    lax.fori_loop(0, CH, zrow, 0)
    for k in range(RPT // CH):
        pltpu.sync_copy(bufs[0], acc.at[pl.ds(s * RPT + k * CH, CH)])
    plsc.subcore_barrier()

    for b in range(LOOK):
        pltpu.async_copy(y_hbm.at[rowv.at[b]], bufs[b], gsems[b])

    def group(g, _):
        for b in range(NBUF):
            j = NBUF * g + b
            bb = (b + LOOK) % NBUF

            @pl.when(j + LOOK < NCH2)
            def _():
                @pl.when(j + LOOK >= NBUF)
                def _():
                    pltpu.make_async_copy(
                        bufs[bb], acc.at[colv.at[j + LOOK - NBUF]],
                        ssems[bb]).wait()
                pltpu.async_copy(y_hbm.at[rowv.at[j + LOOK]], bufs[bb], gsems[bb])

            pltpu.make_async_copy(y_hbm.at[rowv.at[j]], bufs[b], gsems[b]).wait()
            pltpu.async_copy(bufs[b], acc.at[colv.at[j]], ssems[b], add=True)
        return 0
    lax.fori_loop(0, NCH2 // NBUF, group, 0)

    for j in range(NCH2 - NBUF, NCH2):
        pltpu.make_async_copy(bufs[j % NBUF], acc.at[colv.at[j]],
                              ssems[j % NBUF]).wait()

    plsc.subcore_barrier()

    for k in range(RPT // CH):
        r0 = s * RPT + k * CH
        pltpu.sync_copy(acc.at[pl.ds(r0, CH)], bufs[0])
        pltpu.sync_copy(bufs[0], out_hbm.at[c, pl.ds(r0, CH)])


def _mm_body(emb_ref, w_ref, x_ref):
    x_ref[...] = jnp.dot(emb_ref[...], w_ref[...],
                         preferred_element_type=jnp.float32)


def _scale_body(x_ref, hist_ref, y_ref):
    deg = hist_ref[:, 0] + hist_ref[:, 1] + 1.0
    dinv = lax.rsqrt(deg)
    y_ref[...] = x_ref[...] * dinv[:, None]


def _fin_body(p_ref, y_ref, hist_ref, b_ref, o_ref):
    deg = hist_ref[:, 0] + hist_ref[:, 1] + 1.0
    dinv = lax.rsqrt(deg)
    ssum = jnp.concatenate([p_ref[0], p_ref[1]], axis=1) + y_ref[...]
    o_ref[...] = jnp.maximum(ssum * dinv[:, None] + b_ref[...], 0.0)


def kernel(node_emb, edge_index, W, b):
    row = edge_index[0].astype(jnp.int32)
    col = edge_index[1].astype(jnp.int32)
    npd = EP - E
    ar = jnp.arange(npd, dtype=jnp.int32)
    row_flat = jnp.concatenate([row, (ar * 131) % N])
    col_flat = jnp.concatenate([col, N + ar % (NPAD - N)])
    col_p = col_flat.reshape(NW, NCH, CH)
    col16 = col_flat.reshape(16, NCH2, CH)
    row16 = row_flat.reshape(16, NCH2, CH)
    row4 = 2 * row16[None] + jnp.arange(2, dtype=jnp.int32)[:, None, None, None]

    hist = _deg_sc(col_p)
    hist_t = jnp.swapaxes(hist, 0, 1)

    x = pl.pallas_call(
        _mm_body,
        grid=(N // BR,),
        in_specs=[
            pl.BlockSpec((BR, D), lambda i: (i, 0)),
            pl.BlockSpec((D, D), lambda i: (0, 0)),
        ],
        out_specs=pl.BlockSpec((BR, D), lambda i: (i, 0)),
        out_shape=jax.ShapeDtypeStruct((N, D), jnp.float32),
    )(node_emb, W)

    y = pl.pallas_call(
        _scale_body,
        grid=(N // BR,),
        in_specs=[
            pl.BlockSpec((BR, D), lambda i: (i, 0)),
            pl.BlockSpec((BR, 2), lambda i: (i, 0)),
        ],
        out_specs=pl.BlockSpec((BR, D), lambda i: (i, 0)),
        out_shape=jax.ShapeDtypeStruct((N, D), jnp.float32),
    )(x, hist_t)

    y2f = y.reshape(2 * N, DH)
    p = _edge_sc(y2f, row4, col16)

    out = pl.pallas_call(
        _fin_body,
        grid=(N // BR,),
        in_specs=[
            pl.BlockSpec((2, BR, DH), lambda i: (0, i, 0)),
            pl.BlockSpec((BR, D), lambda i: (i, 0)),
            pl.BlockSpec((BR, 2), lambda i: (i, 0)),
            pl.BlockSpec((1, D), lambda i: (0, 0)),
        ],
        out_specs=pl.BlockSpec((BR, D), lambda i: (i, 0)),
        out_shape=jax.ShapeDtypeStruct((N, D), jnp.float32),
    )(p, y, hist_t, b.reshape(1, D))
    return out


# speedup vs baseline: 1.0887x; 1.0197x over previous
import functools

import jax
import jax.numpy as jnp
from jax import lax
from jax.experimental import pallas as pl
from jax.experimental.pallas import tpu as pltpu
from jax.experimental.pallas import tpu_sc as plsc

N = 10000
E = 320000
D = 128
NW = 32
CH = 128
NCH = 80
EPW = NCH * CH
EP = NW * EPW
NPAD = 10240
RPT = NPAD // 16
BR = 2000

_mesh = plsc.VectorSubcoreMesh(core_axis_name="c", subcore_axis_name="s")
_sc_params = pltpu.CompilerParams(use_tc_tiling_on_sc=False)


@functools.partial(
    pl.kernel,
    mesh=_mesh,
    out_type=jax.ShapeDtypeStruct((2, NPAD), jnp.float32),
    scratch_types=[
        pltpu.VMEM((NCH, CH), jnp.int32),
        pltpu.VMEM((CH,), jnp.float32),
        pltpu.VMEM((RPT,), jnp.float32),
        pltpu.VMEM_SHARED((NPAD,), jnp.float32),
        pltpu.SemaphoreType.DMA,
    ],
    compiler_params=_sc_params,
)
def _deg_sc(col_hbm, out_hbm, colv, ones_v, bounce, dacc, sem):
    c = lax.axis_index("c")
    s = lax.axis_index("s")
    wid = c * 16 + s
    pltpu.sync_copy(col_hbm.at[wid], colv)

    def fill_ones(i, _):
        ones_v[pl.ds(i * 16, 16)] = jnp.ones((16,), jnp.float32)
        return 0
    lax.fori_loop(0, CH // 16, fill_ones, 0)

    def fill_zero(i, _):
        bounce[pl.ds(i * 16, 16)] = jnp.zeros((16,), jnp.float32)
        return 0
    lax.fori_loop(0, RPT // 16, fill_zero, 0)

    pltpu.sync_copy(bounce, dacc.at[pl.ds(s * RPT, RPT)])
    plsc.subcore_barrier()

    def body(j, _):
        pltpu.async_copy(ones_v, dacc.at[colv.at[j]], sem, add=True)
        return 0
    lax.fori_loop(0, NCH, body, 0)

    def drain(j, _):
        pltpu.make_async_copy(ones_v, dacc.at[colv.at[j]], sem).wait()
        return 0
    lax.fori_loop(0, NCH, drain, 0)

    plsc.subcore_barrier()
    pltpu.sync_copy(dacc.at[pl.ds(s * RPT, RPT)], bounce)
    pltpu.sync_copy(bounce, out_hbm.at[c, pl.ds(s * RPT, RPT)])


DH = D // 2
NCH2 = 160
EPT = NCH2 * CH


NBUF = 5
LOOK = 3


@functools.partial(
    pl.kernel,
    mesh=_mesh,
    out_type=jax.ShapeDtypeStruct((2, NPAD, DH), jnp.float32),
    scratch_types=[
        pltpu.VMEM((NCH2, CH), jnp.int32),
        pltpu.VMEM((NCH2, CH), jnp.int32),
        [pltpu.VMEM((CH, DH), jnp.float32)] * NBUF,
        [pltpu.SemaphoreType.DMA] * NBUF,
        [pltpu.SemaphoreType.DMA] * NBUF,
        pltpu.VMEM_SHARED((NPAD, DH), jnp.float32),
    ],
    compiler_params=_sc_params,
)
def _edge_sc(y_hbm, row_hbm, col_hbm, out_hbm,
             rowv, colv, bufs, gsems, ssems, acc):
    c = lax.axis_index("c")
    s = lax.axis_index("s")
    pltpu.sync_copy(row_hbm.at[c, s], rowv)
    pltpu.sync_copy(col_hbm.at[s], colv)

    def zrow(i, _):
        def zcol(jj, _2):
            bufs[0][i, pl.ds(jj * 16, 16)] = jnp.zeros((16,), jnp.float32)
            return 0
        lax.fori_loop(0, DH // 16, zcol, 0)
        return 0
    lax.fori_loop(0, CH, zrow, 0)
    for k in range(RPT // CH):
        pltpu.sync_copy(bufs[0], acc.at[pl.ds(s * RPT + k * CH, CH)])
    plsc.subcore_barrier()

    for b in range(LOOK):
        pltpu.async_copy(y_hbm.at[rowv.at[b]], bufs[b], gsems[b])

    def group(g, _):
        for b in range(NBUF):
            j = NBUF * g + b
            bb = (b + LOOK) % NBUF

            @pl.when(j + LOOK < NCH2)
            def _():
                @pl.when(j + LOOK >= NBUF)
                def _():
                    pltpu.make_async_copy(
                        bufs[bb], acc.at[colv.at[j + LOOK - NBUF]],
                        ssems[bb]).wait()
                pltpu.async_copy(y_hbm.at[rowv.at[j + LOOK]], bufs[bb], gsems[bb])

            pltpu.make_async_copy(y_hbm.at[rowv.at[j]], bufs[b], gsems[b]).wait()
            pltpu.async_copy(bufs[b], acc.at[colv.at[j]], ssems[b], add=True)
        return 0
    lax.fori_loop(0, NCH2 // NBUF, group, 0)

    for j in range(NCH2 - NBUF, NCH2):
        pltpu.make_async_copy(bufs[j % NBUF], acc.at[colv.at[j]],
                              ssems[j % NBUF]).wait()

    plsc.subcore_barrier()

    for k in range(RPT // CH):
        r0 = s * RPT + k * CH
        pltpu.sync_copy(acc.at[pl.ds(r0, CH)], bufs[0])
        pltpu.sync_copy(bufs[0], out_hbm.at[c, pl.ds(r0, CH)])


def _mm_body(emb_ref, w_ref, x_ref):
    x_ref[...] = jnp.dot(emb_ref[...], w_ref[...],
                         preferred_element_type=jnp.float32)


def _scale_body(x_ref, hist_ref, y_ref):
    deg = hist_ref[:, 0] + hist_ref[:, 1] + 1.0
    dinv = lax.rsqrt(deg)
    y_ref[...] = x_ref[...] * dinv[:, None]


def _fin_body(p_ref, y_ref, hist_ref, b_ref, o_ref):
    deg = hist_ref[:, 0] + hist_ref[:, 1] + 1.0
    dinv = lax.rsqrt(deg)
    ssum = jnp.concatenate([p_ref[0], p_ref[1]], axis=1) + y_ref[...]
    o_ref[...] = jnp.maximum(ssum * dinv[:, None] + b_ref[...], 0.0)


def kernel(node_emb, edge_index, W, b):
    row = edge_index[0].astype(jnp.int32)
    col = edge_index[1].astype(jnp.int32)
    npd = EP - E
    ar = jnp.arange(npd, dtype=jnp.int32)
    row_flat = jnp.concatenate([row, (ar * 131) % N])
    col_flat = jnp.concatenate([col, N + ar % (NPAD - N)])
    col_p = col_flat.reshape(NW, NCH, CH)
    col16 = col_flat.reshape(16, NCH2, CH)
    row16 = row_flat.reshape(16, NCH2, CH)
    row4 = 2 * row16[None] + jnp.arange(2, dtype=jnp.int32)[:, None, None, None]

    hist = _deg_sc(col_p)
    hist_t = jnp.swapaxes(hist, 0, 1)

    x = pl.pallas_call(
        _mm_body,
        grid=(N // BR,),
        in_specs=[
            pl.BlockSpec((BR, D), lambda i: (i, 0)),
            pl.BlockSpec((D, D), lambda i: (0, 0)),
        ],
        out_specs=pl.BlockSpec((BR, D), lambda i: (i, 0)),
        out_shape=jax.ShapeDtypeStruct((N, D), jnp.float32),
    )(node_emb, W)

    y = pl.pallas_call(
        _scale_body,
        grid=(N // BR,),
        in_specs=[
            pl.BlockSpec((BR, D), lambda i: (i, 0)),
            pl.BlockSpec((BR, 2), lambda i: (i, 0)),
        ],
        out_specs=pl.BlockSpec((BR, D), lambda i: (i, 0)),
        out_shape=jax.ShapeDtypeStruct((N, D), jnp.float32),
    )(x, hist_t)

    y2f = y.reshape(2 * N, DH)
    p = _edge_sc(y2f, row4, col16)

    out = pl.pallas_call(
        _fin_body,
        grid=(N // BR,),
        in_specs=[
            pl.BlockSpec((2, BR, DH), lambda i: (0, i, 0)),
            pl.BlockSpec((BR, D), lambda i: (i, 0)),
            pl.BlockSpec((BR, 2), lambda i: (i, 0)),
            pl.BlockSpec((1, D), lambda i: (0, 0)),
        ],
        out_specs=pl.BlockSpec((BR, D), lambda i: (i, 0)),
        out_shape=jax.ShapeDtypeStruct((N, D), jnp.float32),
    )(p, y, hist_t, b.reshape(1, D))
    return out


# speedup vs baseline: 1.1021x; 1.0123x over previous
import functools

import jax
import jax.numpy as jnp
from jax import lax
from jax.experimental import pallas as pl
from jax.experimental.pallas import tpu as pltpu
from jax.experimental.pallas import tpu_sc as plsc

N = 10000
E = 320000
D = 128
NW = 32
CH = 128
NCH = 80
EPW = NCH * CH
EP = NW * EPW
NPAD = 10240
RPT = NPAD // 16
BR = 2000

_mesh = plsc.VectorSubcoreMesh(core_axis_name="c", subcore_axis_name="s")
_sc_params = pltpu.CompilerParams(use_tc_tiling_on_sc=False)


@functools.partial(
    pl.kernel,
    mesh=_mesh,
    out_type=jax.ShapeDtypeStruct((2, NPAD), jnp.float32),
    scratch_types=[
        pltpu.VMEM((NCH, CH), jnp.int32),
        pltpu.VMEM((CH,), jnp.float32),
        pltpu.VMEM((RPT,), jnp.float32),
        pltpu.VMEM_SHARED((NPAD,), jnp.float32),
        pltpu.SemaphoreType.DMA,
    ],
    compiler_params=_sc_params,
)
def _deg_sc(col_hbm, out_hbm, colv, ones_v, bounce, dacc, sem):
    c = lax.axis_index("c")
    s = lax.axis_index("s")
    wid = c * 16 + s
    pltpu.sync_copy(col_hbm.at[wid], colv)

    def fill_ones(i, _):
        ones_v[pl.ds(i * 16, 16)] = jnp.ones((16,), jnp.float32)
        return 0
    lax.fori_loop(0, CH // 16, fill_ones, 0)

    def fill_zero(i, _):
        bounce[pl.ds(i * 16, 16)] = jnp.zeros((16,), jnp.float32)
        return 0
    lax.fori_loop(0, RPT // 16, fill_zero, 0)

    pltpu.sync_copy(bounce, dacc.at[pl.ds(s * RPT, RPT)])
    plsc.subcore_barrier()

    def body(j, _):
        pltpu.async_copy(ones_v, dacc.at[colv.at[j]], sem, add=True)
        return 0
    lax.fori_loop(0, NCH, body, 0)

    def drain(j, _):
        pltpu.make_async_copy(ones_v, dacc.at[colv.at[j]], sem).wait()
        return 0
    lax.fori_loop(0, NCH, drain, 0)

    plsc.subcore_barrier()
    pltpu.sync_copy(dacc.at[pl.ds(s * RPT, RPT)], bounce)
    pltpu.sync_copy(bounce, out_hbm.at[c, pl.ds(s * RPT, RPT)])


DH = D // 2
NCH2 = 160
EPT = NCH2 * CH


NBUF = 5
LOOK = 4


@functools.partial(
    pl.kernel,
    mesh=_mesh,
    out_type=jax.ShapeDtypeStruct((2, NPAD, DH), jnp.float32),
    scratch_types=[
        pltpu.VMEM((NCH2, CH), jnp.int32),
        pltpu.VMEM((NCH2, CH), jnp.int32),
        [pltpu.VMEM((CH, DH), jnp.float32)] * NBUF,
        [pltpu.SemaphoreType.DMA] * NBUF,
        [pltpu.SemaphoreType.DMA] * NBUF,
        pltpu.VMEM_SHARED((NPAD, DH), jnp.float32),
    ],
    compiler_params=_sc_params,
)
def _edge_sc(y_hbm, row_hbm, col_hbm, out_hbm,
             rowv, colv, bufs, gsems, ssems, acc):
    c = lax.axis_index("c")
    s = lax.axis_index("s")
    pltpu.sync_copy(row_hbm.at[c, s], rowv)
    pltpu.sync_copy(col_hbm.at[s], colv)

    def zrow(i, _):
        def zcol(jj, _2):
            bufs[0][i, pl.ds(jj * 16, 16)] = jnp.zeros((16,), jnp.float32)
            return 0
        lax.fori_loop(0, DH // 16, zcol, 0)
        return 0
    lax.fori_loop(0, CH, zrow, 0)
    for k in range(RPT // CH):
        pltpu.sync_copy(bufs[0], acc.at[pl.ds(s * RPT + k * CH, CH)])
    plsc.subcore_barrier()

    for b in range(LOOK):
        pltpu.async_copy(y_hbm.at[rowv.at[b]], bufs[b], gsems[b])

    def group(g, _):
        for b in range(NBUF):
            j = NBUF * g + b
            bb = (b + LOOK) % NBUF

            @pl.when(j + LOOK < NCH2)
            def _():
                @pl.when(j + LOOK >= NBUF)
                def _():
                    pltpu.make_async_copy(
                        bufs[bb], acc.at[colv.at[j + LOOK - NBUF]],
                        ssems[bb]).wait()
                pltpu.async_copy(y_hbm.at[rowv.at[j + LOOK]], bufs[bb], gsems[bb])

            pltpu.make_async_copy(y_hbm.at[rowv.at[j]], bufs[b], gsems[b]).wait()
            pltpu.async_copy(bufs[b], acc.at[colv.at[j]], ssems[b], add=True)
        return 0
    lax.fori_loop(0, NCH2 // NBUF, group, 0)

    for j in range(NCH2 - NBUF, NCH2):
        pltpu.make_async_copy(bufs[j % NBUF], acc.at[colv.at[j]],
                              ssems[j % NBUF]).wait()

    plsc.subcore_barrier()

    for k in range(RPT // CH):
        r0 = s * RPT + k * CH
        pltpu.sync_copy(acc.at[pl.ds(r0, CH)], bufs[0])
        pltpu.sync_copy(bufs[0], out_hbm.at[c, pl.ds(r0, CH)])


def _mm_body(emb_ref, w_ref, x_ref):
    x_ref[...] = jnp.dot(emb_ref[...], w_ref[...],
                         preferred_element_type=jnp.float32)


def _scale_body(x_ref, hist_ref, y_ref):
    deg = hist_ref[:, 0] + hist_ref[:, 1] + 1.0
    dinv = lax.rsqrt(deg)
    y_ref[...] = x_ref[...] * dinv[:, None]


def _fin_body(p_ref, y_ref, hist_ref, b_ref, o_ref):
    deg = hist_ref[:, 0] + hist_ref[:, 1] + 1.0
    dinv = lax.rsqrt(deg)
    ssum = jnp.concatenate([p_ref[0], p_ref[1]], axis=1) + y_ref[...]
    o_ref[...] = jnp.maximum(ssum * dinv[:, None] + b_ref[...], 0.0)


def kernel(node_emb, edge_index, W, b):
    row = edge_index[0].astype(jnp.int32)
    col = edge_index[1].astype(jnp.int32)
    npd = EP - E
    ar = jnp.arange(npd, dtype=jnp.int32)
    row_flat = jnp.concatenate([row, (ar * 131) % N])
    col_flat = jnp.concatenate([col, N + ar % (NPAD - N)])
    col_p = col_flat.reshape(NW, NCH, CH)
    col16 = col_flat.reshape(16, NCH2, CH)
    row16 = row_flat.reshape(16, NCH2, CH)
    row4 = 2 * row16[None] + jnp.arange(2, dtype=jnp.int32)[:, None, None, None]

    hist = _deg_sc(col_p)
    hist_t = jnp.swapaxes(hist, 0, 1)

    x = pl.pallas_call(
        _mm_body,
        grid=(N // BR,),
        in_specs=[
            pl.BlockSpec((BR, D), lambda i: (i, 0)),
            pl.BlockSpec((D, D), lambda i: (0, 0)),
        ],
        out_specs=pl.BlockSpec((BR, D), lambda i: (i, 0)),
        out_shape=jax.ShapeDtypeStruct((N, D), jnp.float32),
    )(node_emb, W)

    y = pl.pallas_call(
        _scale_body,
        grid=(N // BR,),
        in_specs=[
            pl.BlockSpec((BR, D), lambda i: (i, 0)),
            pl.BlockSpec((BR, 2), lambda i: (i, 0)),
        ],
        out_specs=pl.BlockSpec((BR, D), lambda i: (i, 0)),
        out_shape=jax.ShapeDtypeStruct((N, D), jnp.float32),
    )(x, hist_t)

    y2f = y.reshape(2 * N, DH)
    p = _edge_sc(y2f, row4, col16)

    out = pl.pallas_call(
        _fin_body,
        grid=(N // BR,),
        in_specs=[
            pl.BlockSpec((2, BR, DH), lambda i: (0, i, 0)),
            pl.BlockSpec((BR, D), lambda i: (i, 0)),
            pl.BlockSpec((BR, 2), lambda i: (i, 0)),
            pl.BlockSpec((1, D), lambda i: (0, 0)),
        ],
        out_specs=pl.BlockSpec((BR, D), lambda i: (i, 0)),
        out_shape=jax.ShapeDtypeStruct((N, D), jnp.float32),
    )(p, y, hist_t, b.reshape(1, D))
    return out


# speedup vs baseline: 1.1302x; 1.0255x over previous
import functools

import jax
import jax.numpy as jnp
from jax import lax
from jax.experimental import pallas as pl
from jax.experimental.pallas import tpu as pltpu
from jax.experimental.pallas import tpu_sc as plsc

N = 10000
E = 320000
D = 128
NW = 32
CH = 128
NCH = 80
EPW = NCH * CH
EP = NW * EPW
NPAD = 10240
RPT = NPAD // 16
BR = 2000

_mesh = plsc.VectorSubcoreMesh(core_axis_name="c", subcore_axis_name="s")
_sc_params = pltpu.CompilerParams(use_tc_tiling_on_sc=False)


@functools.partial(
    pl.kernel,
    mesh=_mesh,
    out_type=jax.ShapeDtypeStruct((2, NPAD), jnp.float32),
    scratch_types=[
        pltpu.VMEM((NCH, CH), jnp.int32),
        pltpu.VMEM((CH,), jnp.float32),
        pltpu.VMEM((RPT,), jnp.float32),
        pltpu.VMEM_SHARED((NPAD,), jnp.float32),
        pltpu.SemaphoreType.DMA,
    ],
    compiler_params=_sc_params,
)
def _deg_sc(col_hbm, out_hbm, colv, ones_v, bounce, dacc, sem):
    c = lax.axis_index("c")
    s = lax.axis_index("s")
    wid = c * 16 + s
    pltpu.sync_copy(col_hbm.at[wid], colv)

    def fill_ones(i, _):
        ones_v[pl.ds(i * 16, 16)] = jnp.ones((16,), jnp.float32)
        return 0
    lax.fori_loop(0, CH // 16, fill_ones, 0)

    def fill_zero(i, _):
        bounce[pl.ds(i * 16, 16)] = jnp.zeros((16,), jnp.float32)
        return 0
    lax.fori_loop(0, RPT // 16, fill_zero, 0)

    pltpu.sync_copy(bounce, dacc.at[pl.ds(s * RPT, RPT)])
    plsc.subcore_barrier()

    def body(j, _):
        pltpu.async_copy(ones_v, dacc.at[colv.at[j]], sem, add=True)
        return 0
    lax.fori_loop(0, NCH, body, 0)

    def drain(j, _):
        pltpu.make_async_copy(ones_v, dacc.at[colv.at[j]], sem).wait()
        return 0
    lax.fori_loop(0, NCH, drain, 0)

    plsc.subcore_barrier()
    pltpu.sync_copy(dacc.at[pl.ds(s * RPT, RPT)], bounce)
    pltpu.sync_copy(bounce, out_hbm.at[c, pl.ds(s * RPT, RPT)])


DH = D // 2
NCH2 = 160
EPT = NCH2 * CH


NBUF = 5
LOOK = 4


@functools.partial(
    pl.kernel,
    mesh=_mesh,
    out_type=jax.ShapeDtypeStruct((2, NPAD, DH), jnp.float32),
    scratch_types=[
        pltpu.VMEM((NCH2, CH), jnp.int32),
        pltpu.VMEM((NCH2, CH), jnp.int32),
        [pltpu.VMEM((CH, DH), jnp.float32)] * NBUF,
        [pltpu.SemaphoreType.DMA] * NBUF,
        [pltpu.SemaphoreType.DMA] * NBUF,
        pltpu.VMEM_SHARED((NPAD, DH), jnp.float32),
    ],
    compiler_params=_sc_params,
)
def _edge_sc(y_hbm, row_hbm, col_hbm, out_hbm,
             rowv, colv, bufs, gsems, ssems, acc):
    c = lax.axis_index("c")
    s = lax.axis_index("s")
    pltpu.async_copy(row_hbm.at[c, s], rowv, gsems[0])
    pltpu.async_copy(col_hbm.at[s], colv, gsems[1])

    def zrow(i, _):
        def zcol(jj, _2):
            bufs[0][i, pl.ds(jj * 16, 16)] = jnp.zeros((16,), jnp.float32)
            return 0
        lax.fori_loop(0, DH // 16, zcol, 0)
        return 0
    lax.fori_loop(0, CH, zrow, 0)
    for k in range(RPT // CH):
        pltpu.async_copy(bufs[0], acc.at[pl.ds(s * RPT + k * CH, CH)],
                         ssems[k])
    for k in range(RPT // CH):
        pltpu.make_async_copy(bufs[0], acc.at[pl.ds(s * RPT + k * CH, CH)],
                              ssems[k]).wait()
    pltpu.make_async_copy(row_hbm.at[c, s], rowv, gsems[0]).wait()
    pltpu.make_async_copy(col_hbm.at[s], colv, gsems[1]).wait()
    plsc.subcore_barrier()

    for b in range(LOOK):
        pltpu.async_copy(y_hbm.at[rowv.at[b]], bufs[b], gsems[b])

    def group(g, _):
        for b in range(NBUF):
            j = NBUF * g + b
            bb = (b + LOOK) % NBUF

            @pl.when(j + LOOK < NCH2)
            def _():
                @pl.when(j + LOOK >= NBUF)
                def _():
                    pltpu.make_async_copy(
                        bufs[bb], acc.at[colv.at[j + LOOK - NBUF]],
                        ssems[bb]).wait()
                pltpu.async_copy(y_hbm.at[rowv.at[j + LOOK]], bufs[bb], gsems[bb])

            pltpu.make_async_copy(y_hbm.at[rowv.at[j]], bufs[b], gsems[b]).wait()
            pltpu.async_copy(bufs[b], acc.at[colv.at[j]], ssems[b], add=True)
        return 0
    lax.fori_loop(0, NCH2 // NBUF, group, 0)

    for j in range(NCH2 - NBUF, NCH2):
        pltpu.make_async_copy(bufs[j % NBUF], acc.at[colv.at[j]],
                              ssems[j % NBUF]).wait()

    plsc.subcore_barrier()

    for k in range(RPT // CH):
        r0 = s * RPT + k * CH
        pltpu.sync_copy(acc.at[pl.ds(r0, CH)], bufs[k])
        pltpu.async_copy(bufs[k], out_hbm.at[c, pl.ds(r0, CH)], ssems[k])
    for k in range(RPT // CH):
        r0 = s * RPT + k * CH
        pltpu.make_async_copy(bufs[k], out_hbm.at[c, pl.ds(r0, CH)],
                              ssems[k]).wait()


def _mm_body(emb_ref, w_ref, x_ref):
    x_ref[...] = jnp.dot(emb_ref[...], w_ref[...],
                         preferred_element_type=jnp.float32)


def _scale_body(x_ref, hist_ref, y_ref):
    deg = hist_ref[:, 0] + hist_ref[:, 1] + 1.0
    dinv = lax.rsqrt(deg)
    y_ref[...] = x_ref[...] * dinv[:, None]


def _fin_body(p_ref, y_ref, hist_ref, b_ref, o_ref):
    deg = hist_ref[:, 0] + hist_ref[:, 1] + 1.0
    dinv = lax.rsqrt(deg)
    ssum = jnp.concatenate([p_ref[0], p_ref[1]], axis=1) + y_ref[...]
    o_ref[...] = jnp.maximum(ssum * dinv[:, None] + b_ref[...], 0.0)


def kernel(node_emb, edge_index, W, b):
    row = edge_index[0].astype(jnp.int32)
    col = edge_index[1].astype(jnp.int32)
    npd = EP - E
    ar = jnp.arange(npd, dtype=jnp.int32)
    row_flat = jnp.concatenate([row, (ar * 131) % N])
    col_flat = jnp.concatenate([col, N + ar % (NPAD - N)])
    col_p = col_flat.reshape(NW, NCH, CH)
    col16 = col_flat.reshape(16, NCH2, CH)
    row16 = row_flat.reshape(16, NCH2, CH)
    row4 = 2 * row16[None] + jnp.arange(2, dtype=jnp.int32)[:, None, None, None]

    hist = _deg_sc(col_p)
    hist_t = jnp.swapaxes(hist, 0, 1)

    x = pl.pallas_call(
        _mm_body,
        grid=(N // BR,),
        in_specs=[
            pl.BlockSpec((BR, D), lambda i: (i, 0)),
            pl.BlockSpec((D, D), lambda i: (0, 0)),
        ],
        out_specs=pl.BlockSpec((BR, D), lambda i: (i, 0)),
        out_shape=jax.ShapeDtypeStruct((N, D), jnp.float32),
    )(node_emb, W)

    y = pl.pallas_call(
        _scale_body,
        grid=(N // BR,),
        in_specs=[
            pl.BlockSpec((BR, D), lambda i: (i, 0)),
            pl.BlockSpec((BR, 2), lambda i: (i, 0)),
        ],
        out_specs=pl.BlockSpec((BR, D), lambda i: (i, 0)),
        out_shape=jax.ShapeDtypeStruct((N, D), jnp.float32),
    )(x, hist_t)

    y2f = y.reshape(2 * N, DH)
    p = _edge_sc(y2f, row4, col16)

    out = pl.pallas_call(
        _fin_body,
        grid=(N // BR,),
        in_specs=[
            pl.BlockSpec((2, BR, DH), lambda i: (0, i, 0)),
            pl.BlockSpec((BR, D), lambda i: (i, 0)),
            pl.BlockSpec((BR, 2), lambda i: (i, 0)),
            pl.BlockSpec((1, D), lambda i: (0, 0)),
        ],
        out_specs=pl.BlockSpec((BR, D), lambda i: (i, 0)),
        out_shape=jax.ShapeDtypeStruct((N, D), jnp.float32),
    )(p, y, hist_t, b.reshape(1, D))
    return out


# speedup vs baseline: 1.1315x; 1.0012x over previous
import functools

import jax
import jax.numpy as jnp
from jax import lax
from jax.experimental import pallas as pl
from jax.experimental.pallas import tpu as pltpu
from jax.experimental.pallas import tpu_sc as plsc

N = 10000
E = 320000
D = 128
NW = 32
CH = 128
NCH = 80
EPW = NCH * CH
EP = NW * EPW
NPAD = 10240
RPT = NPAD // 16
BR = 2000

_mesh = plsc.VectorSubcoreMesh(core_axis_name="c", subcore_axis_name="s")
_sc_params = pltpu.CompilerParams(use_tc_tiling_on_sc=False)


@functools.partial(
    pl.kernel,
    mesh=_mesh,
    out_type=jax.ShapeDtypeStruct((2, NPAD), jnp.float32),
    scratch_types=[
        pltpu.VMEM((NCH, CH), jnp.int32),
        pltpu.VMEM((CH,), jnp.float32),
        pltpu.VMEM((RPT,), jnp.float32),
        pltpu.VMEM_SHARED((NPAD,), jnp.float32),
        pltpu.SemaphoreType.DMA,
    ],
    compiler_params=_sc_params,
)
def _deg_sc(col_hbm, out_hbm, colv, ones_v, bounce, dacc, sem):
    c = lax.axis_index("c")
    s = lax.axis_index("s")
    wid = c * 16 + s
    pltpu.async_copy(col_hbm.at[wid], colv, sem)

    def fill_ones(i, _):
        ones_v[pl.ds(i * 16, 16)] = jnp.ones((16,), jnp.float32)
        return 0
    lax.fori_loop(0, CH // 16, fill_ones, 0)

    def fill_zero(i, _):
        bounce[pl.ds(i * 16, 16)] = jnp.zeros((16,), jnp.float32)
        return 0
    lax.fori_loop(0, RPT // 16, fill_zero, 0)

    pltpu.sync_copy(bounce, dacc.at[pl.ds(s * RPT, RPT)])
    pltpu.make_async_copy(col_hbm.at[wid], colv, sem).wait()
    plsc.subcore_barrier()

    def body(j, _):
        pltpu.async_copy(ones_v, dacc.at[colv.at[j]], sem, add=True)
        return 0
    lax.fori_loop(0, NCH, body, 0)

    def drain(j, _):
        pltpu.make_async_copy(ones_v, dacc.at[colv.at[j]], sem).wait()
        return 0
    lax.fori_loop(0, NCH, drain, 0)

    plsc.subcore_barrier()
    pltpu.sync_copy(dacc.at[pl.ds(s * RPT, RPT)], bounce)
    pltpu.sync_copy(bounce, out_hbm.at[c, pl.ds(s * RPT, RPT)])


DH = D // 2
NCH2 = 160
EPT = NCH2 * CH


NBUF = 5
LOOK = 4


@functools.partial(
    pl.kernel,
    mesh=_mesh,
    out_type=jax.ShapeDtypeStruct((2, NPAD, DH), jnp.float32),
    scratch_types=[
        pltpu.VMEM((NCH2, CH), jnp.int32),
        pltpu.VMEM((NCH2, CH), jnp.int32),
        [pltpu.VMEM((CH, DH), jnp.float32)] * NBUF,
        [pltpu.SemaphoreType.DMA] * NBUF,
        [pltpu.SemaphoreType.DMA] * NBUF,
        pltpu.VMEM_SHARED((NPAD, DH), jnp.float32),
    ],
    compiler_params=_sc_params,
)
def _edge_sc(y_hbm, row_hbm, col_hbm, out_hbm,
             rowv, colv, bufs, gsems, ssems, acc):
    c = lax.axis_index("c")
    s = lax.axis_index("s")
    pltpu.async_copy(row_hbm.at[c, s], rowv, gsems[0])
    pltpu.async_copy(col_hbm.at[s], colv, gsems[1])

    def zrow(i, _):
        def zcol(jj, _2):
            bufs[0][i, pl.ds(jj * 16, 16)] = jnp.zeros((16,), jnp.float32)
            return 0
        lax.fori_loop(0, DH // 16, zcol, 0)
        return 0
    lax.fori_loop(0, CH, zrow, 0)
    for k in range(RPT // CH):
        pltpu.async_copy(bufs[0], acc.at[pl.ds(s * RPT + k * CH, CH)],
                         ssems[k])
    for k in range(RPT // CH):
        pltpu.make_async_copy(bufs[0], acc.at[pl.ds(s * RPT + k * CH, CH)],
                              ssems[k]).wait()
    pltpu.make_async_copy(row_hbm.at[c, s], rowv, gsems[0]).wait()
    pltpu.make_async_copy(col_hbm.at[s], colv, gsems[1]).wait()
    plsc.subcore_barrier()

    for b in range(LOOK):
        pltpu.async_copy(y_hbm.at[rowv.at[b]], bufs[b], gsems[b])

    def group(g, _):
        for b in range(NBUF):
            j = NBUF * g + b
            bb = (b + LOOK) % NBUF

            @pl.when(j + LOOK < NCH2)
            def _():
                @pl.when(j + LOOK >= NBUF)
                def _():
                    pltpu.make_async_copy(
                        bufs[bb], acc.at[colv.at[j + LOOK - NBUF]],
                        ssems[bb]).wait()
                pltpu.async_copy(y_hbm.at[rowv.at[j + LOOK]], bufs[bb], gsems[bb])

            pltpu.make_async_copy(y_hbm.at[rowv.at[j]], bufs[b], gsems[b]).wait()
            pltpu.async_copy(bufs[b], acc.at[colv.at[j]], ssems[b], add=True)
        return 0
    lax.fori_loop(0, NCH2 // NBUF, group, 0)

    for j in range(NCH2 - NBUF, NCH2):
        pltpu.make_async_copy(bufs[j % NBUF], acc.at[colv.at[j]],
                              ssems[j % NBUF]).wait()

    plsc.subcore_barrier()

    for k in range(RPT // CH):
        r0 = s * RPT + k * CH
        pltpu.sync_copy(acc.at[pl.ds(r0, CH)], bufs[k])
        pltpu.async_copy(bufs[k], out_hbm.at[c, pl.ds(r0, CH)], ssems[k])
    for k in range(RPT // CH):
        r0 = s * RPT + k * CH
        pltpu.make_async_copy(bufs[k], out_hbm.at[c, pl.ds(r0, CH)],
                              ssems[k]).wait()


def _mm_body(emb_ref, w_ref, x_ref):
    x_ref[...] = jnp.dot(emb_ref[...], w_ref[...],
                         preferred_element_type=jnp.float32)


def _scale_body(x_ref, hist_ref, y_ref):
    deg = hist_ref[:, 0] + hist_ref[:, 1] + 1.0
    dinv = lax.rsqrt(deg)
    y_ref[...] = x_ref[...] * dinv[:, None]


def _fin_body(p_ref, y_ref, hist_ref, b_ref, o_ref):
    deg = hist_ref[:, 0] + hist_ref[:, 1] + 1.0
    dinv = lax.rsqrt(deg)
    ssum = jnp.concatenate([p_ref[0], p_ref[1]], axis=1) + y_ref[...]
    o_ref[...] = jnp.maximum(ssum * dinv[:, None] + b_ref[...], 0.0)


def kernel(node_emb, edge_index, W, b):
    row = edge_index[0].astype(jnp.int32)
    col = edge_index[1].astype(jnp.int32)
    npd = EP - E
    ar = jnp.arange(npd, dtype=jnp.int32)
    row_flat = jnp.concatenate([row, (ar * 131) % N])
    col_flat = jnp.concatenate([col, N + ar % (NPAD - N)])
    col_p = col_flat.reshape(NW, NCH, CH)
    col16 = col_flat.reshape(16, NCH2, CH)
    row16 = row_flat.reshape(16, NCH2, CH)
    row4 = 2 * row16[None] + jnp.arange(2, dtype=jnp.int32)[:, None, None, None]

    hist = _deg_sc(col_p)
    hist_t = jnp.swapaxes(hist, 0, 1)

    x = pl.pallas_call(
        _mm_body,
        grid=(N // BR,),
        in_specs=[
            pl.BlockSpec((BR, D), lambda i: (i, 0)),
            pl.BlockSpec((D, D), lambda i: (0, 0)),
        ],
        out_specs=pl.BlockSpec((BR, D), lambda i: (i, 0)),
        out_shape=jax.ShapeDtypeStruct((N, D), jnp.float32),
    )(node_emb, W)

    y = pl.pallas_call(
        _scale_body,
        grid=(N // BR,),
        in_specs=[
            pl.BlockSpec((BR, D), lambda i: (i, 0)),
            pl.BlockSpec((BR, 2), lambda i: (i, 0)),
        ],
        out_specs=pl.BlockSpec((BR, D), lambda i: (i, 0)),
        out_shape=jax.ShapeDtypeStruct((N, D), jnp.float32),
    )(x, hist_t)

    y2f = y.reshape(2 * N, DH)
    p = _edge_sc(y2f, row4, col16)

    out = pl.pallas_call(
        _fin_body,
        grid=(N // BR,),
        in_specs=[
            pl.BlockSpec((2, BR, DH), lambda i: (0, i, 0)),
            pl.BlockSpec((BR, D), lambda i: (i, 0)),
            pl.BlockSpec((BR, 2), lambda i: (i, 0)),
            pl.BlockSpec((1, D), lambda i: (0, 0)),
        ],
        out_specs=pl.BlockSpec((BR, D), lambda i: (i, 0)),
        out_shape=jax.ShapeDtypeStruct((N, D), jnp.float32),
    )(p, y, hist_t, b.reshape(1, D))
    return out


# speedup vs baseline: 1.1685x; 1.0327x over previous
import functools

import jax
import jax.numpy as jnp
from jax import lax
from jax.experimental import pallas as pl
from jax.experimental.pallas import tpu as pltpu
from jax.experimental.pallas import tpu_sc as plsc

N = 10000
E = 320000
D = 128
NW = 32
CH = 128
NCH = 80
EPW = NCH * CH
EP = NW * EPW
NPAD = 10240
RPT = NPAD // 16
BR = 5000

_mesh = plsc.VectorSubcoreMesh(core_axis_name="c", subcore_axis_name="s")
_sc_params = pltpu.CompilerParams(use_tc_tiling_on_sc=False)


@functools.partial(
    pl.kernel,
    mesh=_mesh,
    out_type=jax.ShapeDtypeStruct((2, NPAD), jnp.float32),
    scratch_types=[
        pltpu.VMEM((NCH, CH), jnp.int32),
        pltpu.VMEM((CH,), jnp.float32),
        pltpu.VMEM((RPT,), jnp.float32),
        pltpu.VMEM_SHARED((NPAD,), jnp.float32),
        pltpu.SemaphoreType.DMA,
    ],
    compiler_params=_sc_params,
)
def _deg_sc(col_hbm, out_hbm, colv, ones_v, bounce, dacc, sem):
    c = lax.axis_index("c")
    s = lax.axis_index("s")
    wid = c * 16 + s
    pltpu.async_copy(col_hbm.at[wid], colv, sem)

    def fill_ones(i, _):
        ones_v[pl.ds(i * 16, 16)] = jnp.ones((16,), jnp.float32)
        return 0
    lax.fori_loop(0, CH // 16, fill_ones, 0)

    def fill_zero(i, _):
        bounce[pl.ds(i * 16, 16)] = jnp.zeros((16,), jnp.float32)
        return 0
    lax.fori_loop(0, RPT // 16, fill_zero, 0)

    pltpu.sync_copy(bounce, dacc.at[pl.ds(s * RPT, RPT)])
    pltpu.make_async_copy(col_hbm.at[wid], colv, sem).wait()
    plsc.subcore_barrier()

    def body(j, _):
        pltpu.async_copy(ones_v, dacc.at[colv.at[j]], sem, add=True)
        return 0
    lax.fori_loop(0, NCH, body, 0)

    def drain(j, _):
        pltpu.make_async_copy(ones_v, dacc.at[colv.at[j]], sem).wait()
        return 0
    lax.fori_loop(0, NCH, drain, 0)

    plsc.subcore_barrier()
    pltpu.sync_copy(dacc.at[pl.ds(s * RPT, RPT)], bounce)
    pltpu.sync_copy(bounce, out_hbm.at[c, pl.ds(s * RPT, RPT)])


DH = D // 2
NCH2 = 160
EPT = NCH2 * CH


NBUF = 5
LOOK = 4


@functools.partial(
    pl.kernel,
    mesh=_mesh,
    out_type=jax.ShapeDtypeStruct((2, NPAD, DH), jnp.float32),
    scratch_types=[
        pltpu.VMEM((NCH2, CH), jnp.int32),
        pltpu.VMEM((NCH2, CH), jnp.int32),
        [pltpu.VMEM((CH, DH), jnp.float32)] * NBUF,
        [pltpu.SemaphoreType.DMA] * NBUF,
        [pltpu.SemaphoreType.DMA] * NBUF,
        pltpu.VMEM_SHARED((NPAD, DH), jnp.float32),
    ],
    compiler_params=_sc_params,
)
def _edge_sc(y_hbm, row_hbm, col_hbm, out_hbm,
             rowv, colv, bufs, gsems, ssems, acc):
    c = lax.axis_index("c")
    s = lax.axis_index("s")
    pltpu.async_copy(row_hbm.at[c, s], rowv, gsems[0])
    pltpu.async_copy(col_hbm.at[s], colv, gsems[1])

    def zrow(i, _):
        def zcol(jj, _2):
            bufs[0][i, pl.ds(jj * 16, 16)] = jnp.zeros((16,), jnp.float32)
            return 0
        lax.fori_loop(0, DH // 16, zcol, 0)
        return 0
    lax.fori_loop(0, CH, zrow, 0)
    for k in range(RPT // CH):
        pltpu.async_copy(bufs[0], acc.at[pl.ds(s * RPT + k * CH, CH)],
                         ssems[k])
    for k in range(RPT // CH):
        pltpu.make_async_copy(bufs[0], acc.at[pl.ds(s * RPT + k * CH, CH)],
                              ssems[k]).wait()
    pltpu.make_async_copy(row_hbm.at[c, s], rowv, gsems[0]).wait()
    pltpu.make_async_copy(col_hbm.at[s], colv, gsems[1]).wait()
    plsc.subcore_barrier()

    for b in range(LOOK):
        pltpu.async_copy(y_hbm.at[rowv.at[b]], bufs[b], gsems[b])

    def group(g, _):
        for b in range(NBUF):
            j = NBUF * g + b
            bb = (b + LOOK) % NBUF

            @pl.when(j + LOOK < NCH2)
            def _():
                @pl.when(j + LOOK >= NBUF)
                def _():
                    pltpu.make_async_copy(
                        bufs[bb], acc.at[colv.at[j + LOOK - NBUF]],
                        ssems[bb]).wait()
                pltpu.async_copy(y_hbm.at[rowv.at[j + LOOK]], bufs[bb], gsems[bb])

            pltpu.make_async_copy(y_hbm.at[rowv.at[j]], bufs[b], gsems[b]).wait()
            pltpu.async_copy(bufs[b], acc.at[colv.at[j]], ssems[b], add=True)
        return 0
    lax.fori_loop(0, NCH2 // NBUF, group, 0)

    for j in range(NCH2 - NBUF, NCH2):
        pltpu.make_async_copy(bufs[j % NBUF], acc.at[colv.at[j]],
                              ssems[j % NBUF]).wait()

    plsc.subcore_barrier()

    for k in range(RPT // CH):
        r0 = s * RPT + k * CH
        pltpu.sync_copy(acc.at[pl.ds(r0, CH)], bufs[k])
        pltpu.async_copy(bufs[k], out_hbm.at[c, pl.ds(r0, CH)], ssems[k])
    for k in range(RPT // CH):
        r0 = s * RPT + k * CH
        pltpu.make_async_copy(bufs[k], out_hbm.at[c, pl.ds(r0, CH)],
                              ssems[k]).wait()


def _mm_body(emb_ref, w_ref, x_ref):
    x_ref[...] = jnp.dot(emb_ref[...], w_ref[...],
                         preferred_element_type=jnp.float32)


def _scale_body(x_ref, hist_ref, y_ref):
    deg = hist_ref[:, 0] + hist_ref[:, 1] + 1.0
    dinv = lax.rsqrt(deg)
    y_ref[...] = x_ref[...] * dinv[:, None]


def _fin_body(p_ref, y_ref, hist_ref, b_ref, o_ref):
    deg = hist_ref[:, 0] + hist_ref[:, 1] + 1.0
    dinv = lax.rsqrt(deg)
    ssum = jnp.concatenate([p_ref[0], p_ref[1]], axis=1) + y_ref[...]
    o_ref[...] = jnp.maximum(ssum * dinv[:, None] + b_ref[...], 0.0)


def kernel(node_emb, edge_index, W, b):
    row = edge_index[0].astype(jnp.int32)
    col = edge_index[1].astype(jnp.int32)
    npd = EP - E
    ar = jnp.arange(npd, dtype=jnp.int32)
    row_flat = jnp.concatenate([row, (ar * 131) % N])
    col_flat = jnp.concatenate([col, N + ar % (NPAD - N)])
    col_p = col_flat.reshape(NW, NCH, CH)
    col16 = col_flat.reshape(16, NCH2, CH)
    row16 = row_flat.reshape(16, NCH2, CH)
    row4 = 2 * row16[None] + jnp.arange(2, dtype=jnp.int32)[:, None, None, None]

    hist = _deg_sc(col_p)
    hist_t = jnp.swapaxes(hist, 0, 1)

    x = pl.pallas_call(
        _mm_body,
        grid=(N // BR,),
        in_specs=[
            pl.BlockSpec((BR, D), lambda i: (i, 0)),
            pl.BlockSpec((D, D), lambda i: (0, 0)),
        ],
        out_specs=pl.BlockSpec((BR, D), lambda i: (i, 0)),
        out_shape=jax.ShapeDtypeStruct((N, D), jnp.float32),
    )(node_emb, W)

    y = pl.pallas_call(
        _scale_body,
        grid=(N // BR,),
        in_specs=[
            pl.BlockSpec((BR, D), lambda i: (i, 0)),
            pl.BlockSpec((BR, 2), lambda i: (i, 0)),
        ],
        out_specs=pl.BlockSpec((BR, D), lambda i: (i, 0)),
        out_shape=jax.ShapeDtypeStruct((N, D), jnp.float32),
    )(x, hist_t)

    y2f = y.reshape(2 * N, DH)
    p = _edge_sc(y2f, row4, col16)

    out = pl.pallas_call(
        _fin_body,
        grid=(N // BR,),
        in_specs=[
            pl.BlockSpec((2, BR, DH), lambda i: (0, i, 0)),
            pl.BlockSpec((BR, D), lambda i: (i, 0)),
            pl.BlockSpec((BR, 2), lambda i: (i, 0)),
            pl.BlockSpec((1, D), lambda i: (0, 0)),
        ],
        out_specs=pl.BlockSpec((BR, D), lambda i: (i, 0)),
        out_shape=jax.ShapeDtypeStruct((N, D), jnp.float32),
    )(p, y, hist_t, b.reshape(1, D))
    return out
